# baseline (speedup 1.0000x reference)
import jax
import jax.numpy as jnp
from jax import lax
from jax.experimental import pallas as pl
from jax.experimental.pallas import tpu as pltpu


def _pick_tile(hw):
    for cand in (512, 256, 128):
        if hw % cand == 0 and hw > cand:
            return cand
    return hw


def _make_body(H, W, cin, O, tile, nb):
    HW = H * W
    C = cin + 2
    n_taps = 9

    def _body(x_ref, w_om_ref, b_om_ref, w_dcn_ref, b_dcn_ref,
              out_ref, idx_ref, mask_ref, xf_ref, wall_ref, wrow_ref):
        f32 = jnp.float32
        bf16 = jnp.bfloat16

        @pl.when(pl.program_id(0) == 0)
        def _prologue():
            wall_ref[...] = jnp.zeros((n_taps * 32, C), f32)
            i_io = lax.broadcasted_iota(jnp.int32, (C * n_taps, C), 0)
            c_io = lax.broadcasted_iota(jnp.int32, (C * n_taps, C), 1)
            for k in range(n_taps):
                pk = jnp.where(i_io == c_io * n_taps + k, 1.0, 0.0)
                wall_ref[k * 32:k * 32 + 3 * n_taps, :] = jnp.dot(
                    w_om_ref[...], pk, preferred_element_type=f32)
                wrow_ref[:, k * C:(k + 1) * C] = jnp.dot(
                    w_dcn_ref[...], pk,
                    preferred_element_type=f32).astype(bf16)

        pos = lax.broadcasted_iota(jnp.int32, (1, HW), 1)
        r = pos // W
        c = pos - r * W
        hof = r.astype(f32)
        wof = c.astype(f32)

        xf_ref[cin:cin + 1, :] = hof * (1.0 / H)
        xf_ref[cin + 1:C, :] = wof * (1.0 / W) - 0.5

        rr18 = lax.broadcasted_iota(jnp.int32, (2 * n_taps, HW), 0)
        k9 = jnp.where(rr18 < n_taps, rr18, rr18 - n_taps)
        base = jnp.where(
            rr18 < n_taps,
            hof * (1.0 / H) + (k9 // 3 - 1).astype(f32),
            wof * (1.0 / W) - 0.5 + (k9 - (k9 // 3) * 3 - 1).astype(f32))

        for ib in range(nb):
            xf_ref[0:cin, :] = x_ref[ib]
            x = xf_ref[...]

            y_all = jnp.dot(wall_ref[...], x, preferred_element_type=f32)
            om = jnp.zeros((32, HW), f32)
            for k in range(n_taps):
                dy, dx = k // 3 - 1, k % 3 - 1
                s = dy * W + dx
                yk = y_all[k * 32:(k + 1) * 32, :]
                rolled = pltpu.roll(yk, (-s) % HW, axis=1) if s else yk
                valid = ((r + dy >= 0) & (r + dy < H)
                         & (c + dx >= 0) & (c + dx < W))
                om = om + jnp.where(valid, rolled, 0.0)

            offset = om[:2 * n_taps] + b_om_ref[0:2 * n_taps]
            maskv = jax.nn.sigmoid(om[2 * n_taps:3 * n_taps]
                                   + b_om_ref[2 * n_taps:3 * n_taps])
            idx_ref[ib] = base + offset
            mask_ref[ib] = maskv

            xb = x.astype(bf16)
            for t0 in range(0, HW, tile):
                rT = lax.broadcasted_iota(jnp.int32, (H, tile), 0).astype(f32)
                zs = []
                for k in range(n_taps):
                    i, j = k // 3, k % 3
                    py = hof[:, t0:t0 + tile] + float(i - 1) \
                        + offset[2 * k:2 * k + 1, t0:t0 + tile]
                    px = wof[:, t0:t0 + tile] + float(j - 1) \
                        + offset[2 * k + 1:2 * k + 2, t0:t0 + tile]
                    wy = (jnp.maximum(1.0 - jnp.abs(rT - py), 0.0)
                          * maskv[k:k + 1, t0:t0 + tile]).astype(bf16)
                    wx = jnp.maximum(1.0 - jnp.abs(rT - px), 0.0).astype(bf16)
                    sk = (wy.reshape(H, 1, tile)
                          * wx.reshape(1, W, tile)).reshape(HW, tile)
                    zs.append(jnp.dot(xb, sk,
                                      preferred_element_type=f32).astype(bf16))
                zcat = jnp.concatenate(zs, axis=0)
                acc = jnp.dot(wrow_ref[...], zcat, preferred_element_type=f32)
                out_ref[ib, :, t0:t0 + tile] = acc + b_dcn_ref[...]

    return _body


def kernel(x, weight, bias, conv_offset_mask_weight, conv_offset_mask_bias):
    B, cin, H, W = x.shape
    C = cin + 2
    O = weight.shape[0]
    HW = H * W
    n_taps = 9
    tile = _pick_tile(HW)
    nb = 4
    while B % nb:
        nb //= 2

    x_cm = x.reshape(B, cin, HW)
    w_om_flat = conv_offset_mask_weight.reshape(3 * n_taps, C * n_taps)
    b_om_col = conv_offset_mask_bias.reshape(3 * n_taps, 1)
    w_dcn_flat = weight.reshape(O, C * n_taps)
    b_dcn_col = bias.reshape(O, 1)

    body = _make_body(H, W, cin, O, tile, nb)
    out, idx, maskv = pl.pallas_call(
        body,
        grid=(B // nb,),
        in_specs=[
            pl.BlockSpec((nb, cin, HW), lambda b: (b, 0, 0)),
            pl.BlockSpec((3 * n_taps, C * n_taps), lambda b: (0, 0)),
            pl.BlockSpec((3 * n_taps, 1), lambda b: (0, 0)),
            pl.BlockSpec((O, C * n_taps), lambda b: (0, 0)),
            pl.BlockSpec((O, 1), lambda b: (0, 0)),
        ],
        out_specs=[
            pl.BlockSpec((nb, O, HW), lambda b: (b, 0, 0)),
            pl.BlockSpec((nb, 2 * n_taps, HW), lambda b: (b, 0, 0)),
            pl.BlockSpec((nb, n_taps, HW), lambda b: (b, 0, 0)),
        ],
        out_shape=[
            jax.ShapeDtypeStruct((B, O, HW), jnp.float32),
            jax.ShapeDtypeStruct((B, 2 * n_taps, HW), jnp.float32),
            jax.ShapeDtypeStruct((B, n_taps, HW), jnp.float32),
        ],
        scratch_shapes=[
            pltpu.VMEM((C, HW), jnp.float32),
            pltpu.VMEM((n_taps * 32, C), jnp.float32),
            pltpu.VMEM((O, n_taps * C), jnp.bfloat16),
        ],
        compiler_params=pltpu.CompilerParams(
            dimension_semantics=("arbitrary",),
            vmem_limit_bytes=64 * 1024 * 1024),
    )(x_cm, w_om_flat, b_om_col, w_dcn_flat, b_dcn_col)

    return (out.reshape(B, O, H, W),
            idx.reshape(B, 2 * n_taps, H, W),
            maskv.reshape(B, n_taps, H, W))


# speedup vs baseline: 1.0766x; 1.0766x over previous
import jax
import jax.numpy as jnp
from jax import lax
from jax.experimental import pallas as pl
from jax.experimental.pallas import tpu as pltpu


def _pick_tile(hw):
    for cand in (512, 256, 128):
        if hw % cand == 0 and hw > cand:
            return cand
    return hw


def _make_body(H, W, cin, O, tile, nb):
    HW = H * W
    C = cin + 2
    n_taps = 9

    def _body(x_ref, w_om_ref, b_om_ref, w_dcn_ref, b_dcn_ref,
              out_ref, idx_ref, mask_ref, xf_ref, wall_ref, wdcn_ref):
        f32 = jnp.float32
        bf16 = jnp.bfloat16

        @pl.when(pl.program_id(0) == 0)
        def _prologue():
            wall_ref[...] = jnp.zeros((n_taps * 32, C), f32)
            i_io = lax.broadcasted_iota(jnp.int32, (C * n_taps, C), 0)
            c_io = lax.broadcasted_iota(jnp.int32, (C * n_taps, C), 1)
            for k in range(n_taps):
                pk = jnp.where(i_io == c_io * n_taps + k, 1.0, 0.0)
                wall_ref[k * 32:k * 32 + 3 * n_taps, :] = jnp.dot(
                    w_om_ref[...], pk, preferred_element_type=f32)
                wdcn_ref[k * O:(k + 1) * O, :] = jnp.dot(
                    w_dcn_ref[...], pk,
                    preferred_element_type=f32).astype(bf16)

        pos = lax.broadcasted_iota(jnp.int32, (1, HW), 1)
        r = pos // W
        c = pos - r * W
        hof = r.astype(f32)
        wof = c.astype(f32)

        xf_ref[cin:cin + 1, :] = hof * (1.0 / H)
        xf_ref[cin + 1:C, :] = wof * (1.0 / W) - 0.5

        rr18 = lax.broadcasted_iota(jnp.int32, (2 * n_taps, HW), 0)
        k9 = jnp.where(rr18 < n_taps, rr18, rr18 - n_taps)
        base = jnp.where(
            rr18 < n_taps,
            hof * (1.0 / H) + (k9 // 3 - 1).astype(f32),
            wof * (1.0 / W) - 0.5 + (k9 - (k9 // 3) * 3 - 1).astype(f32))

        for ib in range(nb):
            xf_ref[0:cin, :] = x_ref[ib]
            x = xf_ref[...]

            y_all = jnp.dot(wall_ref[...], x, preferred_element_type=f32)
            om = jnp.zeros((32, HW), f32)
            for k in range(n_taps):
                dy, dx = k // 3 - 1, k % 3 - 1
                s = dy * W + dx
                yk = y_all[k * 32:(k + 1) * 32, :]
                rolled = pltpu.roll(yk, (-s) % HW, axis=1) if s else yk
                valid = ((r + dy >= 0) & (r + dy < H)
                         & (c + dx >= 0) & (c + dx < W))
                om = om + jnp.where(valid, rolled, 0.0)

            offset = om[:2 * n_taps] + b_om_ref[0:2 * n_taps]
            maskv = jax.nn.sigmoid(om[2 * n_taps:3 * n_taps]
                                   + b_om_ref[2 * n_taps:3 * n_taps])
            idx_ref[ib] = base + offset
            mask_ref[ib] = maskv

            xw = jnp.dot(wdcn_ref[...], x.astype(bf16),
                         preferred_element_type=f32).astype(bf16)

            for t0 in range(0, HW, tile):
                rT = lax.broadcasted_iota(jnp.int32, (H, tile), 0).astype(f32)
                acc = jnp.zeros((O, tile), f32)
                for k in range(n_taps):
                    i, j = k // 3, k % 3
                    py = hof[:, t0:t0 + tile] + float(i - 1) \
                        + offset[2 * k:2 * k + 1, t0:t0 + tile]
                    px = wof[:, t0:t0 + tile] + float(j - 1) \
                        + offset[2 * k + 1:2 * k + 2, t0:t0 + tile]
                    wy = (jnp.maximum(1.0 - jnp.abs(rT - py), 0.0)
                          * maskv[k:k + 1, t0:t0 + tile]).astype(bf16)
                    wx = jnp.maximum(1.0 - jnp.abs(rT - px), 0.0).astype(bf16)
                    sk = (wy.reshape(H, 1, tile)
                          * wx.reshape(1, W, tile)).reshape(HW, tile)
                    acc = acc + jnp.dot(xw[k * O:(k + 1) * O], sk,
                                        preferred_element_type=f32)
                out_ref[ib, :, t0:t0 + tile] = acc + b_dcn_ref[...]

    return _body


def kernel(x, weight, bias, conv_offset_mask_weight, conv_offset_mask_bias):
    B, cin, H, W = x.shape
    C = cin + 2
    O = weight.shape[0]
    HW = H * W
    n_taps = 9
    tile = _pick_tile(HW)
    nb = 8
    while B % nb:
        nb //= 2

    x_cm = x.reshape(B, cin, HW)
    w_om_flat = conv_offset_mask_weight.reshape(3 * n_taps, C * n_taps)
    b_om_col = conv_offset_mask_bias.reshape(3 * n_taps, 1)
    w_dcn_flat = weight.reshape(O, C * n_taps)
    b_dcn_col = bias.reshape(O, 1)

    body = _make_body(H, W, cin, O, tile, nb)
    out, idx, maskv = pl.pallas_call(
        body,
        grid=(B // nb,),
        in_specs=[
            pl.BlockSpec((nb, cin, HW), lambda b: (b, 0, 0)),
            pl.BlockSpec((3 * n_taps, C * n_taps), lambda b: (0, 0)),
            pl.BlockSpec((3 * n_taps, 1), lambda b: (0, 0)),
            pl.BlockSpec((O, C * n_taps), lambda b: (0, 0)),
            pl.BlockSpec((O, 1), lambda b: (0, 0)),
        ],
        out_specs=[
            pl.BlockSpec((nb, O, HW), lambda b: (b, 0, 0)),
            pl.BlockSpec((nb, 2 * n_taps, HW), lambda b: (b, 0, 0)),
            pl.BlockSpec((nb, n_taps, HW), lambda b: (b, 0, 0)),
        ],
        out_shape=[
            jax.ShapeDtypeStruct((B, O, HW), jnp.float32),
            jax.ShapeDtypeStruct((B, 2 * n_taps, HW), jnp.float32),
            jax.ShapeDtypeStruct((B, n_taps, HW), jnp.float32),
        ],
        scratch_shapes=[
            pltpu.VMEM((C, HW), jnp.float32),
            pltpu.VMEM((n_taps * 32, C), jnp.float32),
            pltpu.VMEM((n_taps * O, C), jnp.bfloat16),
        ],
        compiler_params=pltpu.CompilerParams(
            dimension_semantics=("arbitrary",),
            vmem_limit_bytes=64 * 1024 * 1024),
    )(x_cm, w_om_flat, b_om_col, w_dcn_flat, b_dcn_col)

    return (out.reshape(B, O, H, W),
            idx.reshape(B, 2 * n_taps, H, W),
            maskv.reshape(B, n_taps, H, W))
